# baseline (speedup 1.0000x reference)
import functools

import jax
import jax.numpy as jnp
from jax import lax
from jax.experimental import pallas as pl
from jax.experimental.pallas import tpu as pltpu
from jax.experimental.pallas import tpu_sc as plsc

_N_ATTR = 26
_N_VALUES = 1000
_EMBED_DIM = 128
_N_HIDDEN = 1024
_BATCH = 4096

_NC = 2
_NS = 16
_NW = _NC * _NS

_CH = 128
_ROWS = _BATCH * _N_ATTR
_RPW = _ROWS // _NW
_NCH = _RPW // _CH


@functools.cache
def _build_gather_sc():
    mesh = plsc.VectorSubcoreMesh(
        core_axis_name="c", subcore_axis_name="s", num_cores=_NC, num_subcores=_NS
    )

    @functools.partial(
        pl.kernel,
        out_type=jax.ShapeDtypeStruct((_N_ATTR, _BATCH, _EMBED_DIM), jnp.float32),
        mesh=mesh,
        scratch_types=[
            pltpu.VMEM((_NCH, _CH), jnp.int32),
            pltpu.VMEM((2, _CH, _EMBED_DIM), jnp.float32),
            pltpu.SemaphoreType.DMA,
            pltpu.SemaphoreType.DMA,
            pltpu.SemaphoreType.DMA,
            pltpu.SemaphoreType.DMA,
        ],
    )
    def _gather_sc(idx_hbm, table_hbm, out_hbm, idx_v, rows_v, g0, g1, w0, w1):
        wid = lax.axis_index("s") * _NC + lax.axis_index("c")
        pltpu.sync_copy(idx_hbm.at[wid], idx_v)
        gsems = (g0, g1)
        wsems = (w0, w1)
        base = wid * _RPW
        gathers = [None] * _NCH
        writes = [None] * _NCH
        gathers[0] = pltpu.async_copy(table_hbm.at[idx_v.at[0]], rows_v.at[0], gsems[0])
        for j in range(_NCH):
            b = j & 1
            gathers[j].wait()
            if j >= 1:
                writes[j - 1].wait()
            if j + 1 < _NCH:
                gathers[j + 1] = pltpu.async_copy(
                    table_hbm.at[idx_v.at[j + 1]], rows_v.at[b ^ 1], gsems[b ^ 1]
                )
            r0 = base + j * _CH
            writes[j] = pltpu.async_copy(
                rows_v.at[b],
                out_hbm.at[r0 // _BATCH, pl.ds(r0 % _BATCH, _CH)],
                wsems[b],
            )
        writes[_NCH - 1].wait()

    return _gather_sc


def _mm_body(e_ref, w_ref, b_ref, o_ref):
    a = pl.program_id(0)
    acc = jnp.dot(e_ref[0], w_ref[0], preferred_element_type=jnp.float32)

    @pl.when(a == 0)
    def _init():
        o_ref[...] = acc + b_ref[...]

    @pl.when(a > 0)
    def _accum():
        o_ref[...] += acc


def _matmul_tc(emb, fc_w3, fc_b2d):
    return pl.pallas_call(
        _mm_body,
        grid=(_N_ATTR,),
        in_specs=[
            pl.BlockSpec((1, _BATCH, _EMBED_DIM), lambda a: (a, 0, 0)),
            pl.BlockSpec((1, _EMBED_DIM, _N_HIDDEN), lambda a: (a, 0, 0)),
            pl.BlockSpec((1, _N_HIDDEN), lambda a: (0, 0)),
        ],
        out_specs=pl.BlockSpec((_BATCH, _N_HIDDEN), lambda a: (0, 0)),
        out_shape=jax.ShapeDtypeStruct((_BATCH, _N_HIDDEN), jnp.float32),
    )(emb, fc_w3, fc_b2d)


def kernel(x, table, fc_w, fc_b):
    offs = (jnp.arange(_N_ATTR, dtype=jnp.int32) * _N_VALUES)[:, None]
    idx = (x.astype(jnp.int32).T + offs).reshape(_NW, _NCH, _CH)
    emb = _build_gather_sc()(idx, table)
    fc_w3 = fc_w.reshape(_N_ATTR, _EMBED_DIM, _N_HIDDEN)
    return _matmul_tc(emb, fc_w3, fc_b.reshape(1, _N_HIDDEN))


# speedup vs baseline: 1.3258x; 1.3258x over previous
import functools

import jax
import jax.numpy as jnp
from jax import lax
from jax.experimental import pallas as pl
from jax.experimental.pallas import tpu as pltpu
from jax.experimental.pallas import tpu_sc as plsc

_N_ATTR = 26
_N_VALUES = 1000
_EMBED_DIM = 128
_N_HIDDEN = 1024
_BATCH = 4096

_NC = 2
_NS = 16
_NW = _NC * _NS

_CH = 128
_ROWS = _BATCH * _N_ATTR
_RPW = _ROWS // _NW
_NCH = _RPW // _CH


@functools.cache
def _build_gather_sc():
    mesh = plsc.VectorSubcoreMesh(
        core_axis_name="c", subcore_axis_name="s", num_cores=_NC, num_subcores=_NS
    )

    @functools.partial(
        pl.kernel,
        out_type=jax.ShapeDtypeStruct((_N_ATTR, _BATCH, _EMBED_DIM), jnp.float32),
        mesh=mesh,
        scratch_types=[
            pltpu.VMEM((_NCH, _CH), jnp.int32),
            pltpu.VMEM((2, _CH, _EMBED_DIM), jnp.float32),
            pltpu.SemaphoreType.DMA,
            pltpu.SemaphoreType.DMA,
            pltpu.SemaphoreType.DMA,
            pltpu.SemaphoreType.DMA,
        ],
    )
    def _gather_sc(idx_hbm, table_hbm, out_hbm, idx_v, rows_v, g0, g1, w0, w1):
        wid = lax.axis_index("s") * _NC + lax.axis_index("c")
        pltpu.sync_copy(idx_hbm.at[wid], idx_v)
        gsems = (g0, g1)
        wsems = (w0, w1)
        base = wid * _RPW
        gathers = [None] * _NCH
        writes = [None] * _NCH
        gathers[0] = pltpu.async_copy(table_hbm.at[idx_v.at[0]], rows_v.at[0], gsems[0])
        for j in range(_NCH):
            b = j & 1
            gathers[j].wait()
            if j >= 1:
                writes[j - 1].wait()
            if j + 1 < _NCH:
                gathers[j + 1] = pltpu.async_copy(
                    table_hbm.at[idx_v.at[j + 1]], rows_v.at[b ^ 1], gsems[b ^ 1]
                )
            r0 = base + j * _CH
            writes[j] = pltpu.async_copy(
                rows_v.at[b],
                out_hbm.at[r0 // _BATCH, pl.ds(r0 % _BATCH, _CH)],
                wsems[b],
            )
        writes[_NCH - 1].wait()

    return _gather_sc


_BM = 512


def _mm_body(e_ref, w_ref, b_ref, o_ref):
    acc = b_ref[...].astype(jnp.float32)
    for a in range(_N_ATTR):
        acc = acc + jnp.dot(
            e_ref[a], w_ref[a], preferred_element_type=jnp.float32
        )
    o_ref[...] = acc


def _matmul_tc(emb, fc_w3, fc_b2d):
    return pl.pallas_call(
        _mm_body,
        grid=(_BATCH // _BM,),
        in_specs=[
            pl.BlockSpec((_N_ATTR, _BM, _EMBED_DIM), lambda i: (0, i, 0)),
            pl.BlockSpec((_N_ATTR, _EMBED_DIM, _N_HIDDEN), lambda i: (0, 0, 0)),
            pl.BlockSpec((1, _N_HIDDEN), lambda i: (0, 0)),
        ],
        out_specs=pl.BlockSpec((_BM, _N_HIDDEN), lambda i: (i, 0)),
        out_shape=jax.ShapeDtypeStruct((_BATCH, _N_HIDDEN), jnp.float32),
    )(emb, fc_w3, fc_b2d)


def kernel(x, table, fc_w, fc_b):
    offs = (jnp.arange(_N_ATTR, dtype=jnp.int32) * _N_VALUES)[:, None]
    idx = (x.astype(jnp.int32).T + offs).reshape(_NW, _NCH, _CH)
    emb = _build_gather_sc()(idx, table)
    fc_w3 = fc_w.reshape(_N_ATTR, _EMBED_DIM, _N_HIDDEN)
    return _matmul_tc(emb, fc_w3, fc_b.reshape(1, _N_HIDDEN))


# speedup vs baseline: 1.6617x; 1.2533x over previous
import functools

import jax
import jax.numpy as jnp
from jax import lax
from jax.experimental import pallas as pl
from jax.experimental.pallas import tpu as pltpu
from jax.experimental.pallas import tpu_sc as plsc

_N_ATTR = 26
_N_VALUES = 1000
_EMBED_DIM = 128
_N_HIDDEN = 1024
_BATCH = 4096
_K = _N_ATTR * _EMBED_DIM

_NC = 2
_NS = 16
_NW = _NC * _NS

_CH = 128
_ROWS = _BATCH * _N_ATTR
_RPW = _ROWS // _NW
_NCH = _RPW // _CH


@functools.cache
def _build_gather_sc():
    mesh = plsc.VectorSubcoreMesh(
        core_axis_name="c", subcore_axis_name="s", num_cores=_NC, num_subcores=_NS
    )

    @functools.partial(
        pl.kernel,
        out_type=jax.ShapeDtypeStruct((_BATCH, _K), jnp.float32),
        mesh=mesh,
        scratch_types=[
            pltpu.VMEM((_NCH, _CH), jnp.int32),
            pltpu.VMEM((2, _CH, _EMBED_DIM), jnp.float32),
            pltpu.SemaphoreType.DMA,
            pltpu.SemaphoreType.DMA,
            pltpu.SemaphoreType.DMA,
            pltpu.SemaphoreType.DMA,
        ],
    )
    def _gather_sc(idx_hbm, table_hbm, out_hbm, idx_v, rows_v, g0, g1, w0, w1):
        wid = lax.axis_index("s") * _NC + lax.axis_index("c")
        pltpu.sync_copy(idx_hbm.at[wid], idx_v)
        gsems = (g0, g1)
        wsems = (w0, w1)
        base = wid * _RPW
        gathers = [None] * _NCH
        writes = [None] * _NCH
        gathers[0] = pltpu.async_copy(table_hbm.at[idx_v.at[0]], rows_v.at[0], gsems[0])
        for j in range(_NCH):
            b = j & 1
            gathers[j].wait()
            if j >= 1:
                writes[j - 1].wait()
            if j + 1 < _NCH:
                gathers[j + 1] = pltpu.async_copy(
                    table_hbm.at[idx_v.at[j + 1]], rows_v.at[b ^ 1], gsems[b ^ 1]
                )
            r0 = base + j * _CH
            writes[j] = pltpu.async_copy(
                rows_v.at[b],
                out_hbm.at[pl.ds(r0 % _BATCH, _CH), pl.ds((r0 // _BATCH) * _EMBED_DIM, _EMBED_DIM)],
                wsems[b],
            )
        writes[_NCH - 1].wait()

    return _gather_sc


_BM = 512


def _mm_body(a_ref, w_ref, b_ref, o_ref):
    o_ref[...] = (
        jnp.dot(a_ref[...], w_ref[...], preferred_element_type=jnp.float32)
        + b_ref[...]
    )


def _matmul_tc(flat, fc_w, fc_b2d):
    return pl.pallas_call(
        _mm_body,
        grid=(_BATCH // _BM,),
        in_specs=[
            pl.BlockSpec((_BM, _K), lambda i: (i, 0)),
            pl.BlockSpec((_K, _N_HIDDEN), lambda i: (0, 0)),
            pl.BlockSpec((1, _N_HIDDEN), lambda i: (0, 0)),
        ],
        out_specs=pl.BlockSpec((_BM, _N_HIDDEN), lambda i: (i, 0)),
        out_shape=jax.ShapeDtypeStruct((_BATCH, _N_HIDDEN), jnp.float32),
    )(flat, fc_w, fc_b2d)


def kernel(x, table, fc_w, fc_b):
    offs = (jnp.arange(_N_ATTR, dtype=jnp.int32) * _N_VALUES)[:, None]
    idx = (x.astype(jnp.int32).T + offs).reshape(_NW, _NCH, _CH)
    flat = _build_gather_sc()(idx, table)
    return _matmul_tc(flat, fc_w, fc_b.reshape(1, _N_HIDDEN))
